# baseline (speedup 1.0000x reference)
import functools
import math

import numpy as np
import jax
import jax.numpy as jnp
from jax import lax
from jax.experimental import pallas as pl
from jax.experimental.pallas import tpu as pltpu
from jax.experimental.pallas import tpu_sc as plsc

EMBED_DIMS = 32
_HALF = EMBED_DIMS // 2


def _make_sc_scatter(B: int, L: int, P: int):
    info = plsc.get_sparse_core_info()
    NW = info.num_cores * info.num_subcores
    assert B % NW == 0
    rows_w = B // NW
    blk_len = rows_w * L
    pk_w = rows_w * P
    assert pk_w % 16 == 0 and pk_w % 8 == 0 and blk_len % 8 == 0
    n_vec = pk_w // 16

    mesh = plsc.VectorSubcoreMesh(core_axis_name="c", subcore_axis_name="s")

    @functools.partial(
        pl.kernel,
        out_type=jax.ShapeDtypeStruct((B * L,), jnp.float32),
        mesh=mesh,
        compiler_params=pltpu.CompilerParams(needs_layout_passes=False),
        scratch_types=[
            pltpu.VMEM((blk_len,), jnp.float32),
            pltpu.VMEM((pk_w,), jnp.int32),
            pltpu.VMEM((pk_w,), jnp.int32),
        ],
    )
    def sc_scatter(peaks_hbm, rowoff_hbm, out_hbm, blk, pk, ro):
        wid = lax.axis_index("s") * info.num_cores + lax.axis_index("c")

        pltpu.sync_copy(peaks_hbm.at[pl.ds(wid * pk_w, pk_w)], pk)
        pltpu.sync_copy(rowoff_hbm.at[pl.ds(wid * pk_w, pk_w)], ro)

        z16 = jnp.zeros((16,), jnp.float32)

        def zero_body(i, carry):
            base = i * 128
            for j in range(8):
                blk[pl.ds(base + j * 16, 16)] = z16
            return carry

        lax.fori_loop(0, blk_len // 128, zero_body, 0)

        ones16 = jnp.ones((16,), jnp.float32)

        def scat_body(j, carry):
            col = pk[pl.ds(j * 16, 16)]
            off = ro[pl.ds(j * 16, 16)]
            plsc.store_scatter(blk, [off + col], ones16, mask=col != 0)
            return carry

        lax.fori_loop(0, n_vec, scat_body, 0)

        pltpu.sync_copy(blk, out_hbm.at[pl.ds(wid * blk_len, blk_len)])

    return sc_scatter, rows_w


def _tc_body(se_ref, so_ref, de_ref, do_ref, v_ref, w_ref, b_ref, o_ref):
    RB, LB2 = se_ref.shape
    OUT = o_ref.shape[2] // 2
    feat = OUT - EMBED_DIMS - 1
    k16 = lax.broadcasted_iota(jnp.int32, (1, _HALF), 1).astype(jnp.float32)
    log_step = math.log(1000.0) / (_HALF - 1)
    ang_c = (2.0 * math.pi) * jnp.exp(k16 * log_step)
    lane = lax.broadcasted_iota(jnp.int32, (LB2, 2 * OUT), 1)
    av = v_ref[...] * ang_c
    row_all = b_ref[...][0:1, :] + jnp.concatenate(
        [jnp.zeros((RB, EMBED_DIMS), jnp.float32), jnp.sin(av), jnp.cos(av)],
        axis=1)
    row_all2 = jnp.concatenate([row_all, row_all], axis=1)
    sd2 = jnp.concatenate([lax.transpose(se_ref[...], (1, 0)),
                           lax.transpose(so_ref[...], (1, 0))], axis=1)
    dd2 = jnp.concatenate([lax.transpose(de_ref[...], (1, 0)),
                           lax.transpose(do_ref[...], (1, 0))], axis=1)
    is31 = (lane % OUT) == feat
    wv2 = jnp.broadcast_to(
        jnp.where((lax.broadcasted_iota(jnp.int32, (1, 2 * OUT), 1) % OUT)
                  == feat,
                  jnp.float32(1.0),
                  jnp.concatenate([w_ref[...][0:1, :], w_ref[...][0:1, :]],
                                  axis=1)), (LB2, 2 * OUT))
    half = (lane // OUT) * RB
    for r in range(RB):
        idx = half + r
        mixed_s = jnp.take_along_axis(sd2, idx, axis=1)
        mixed_d = jnp.take_along_axis(dd2, idx, axis=1)
        mixed = jnp.where(is31, mixed_d, mixed_s)
        rowv = jnp.broadcast_to(row_all2[r:r + 1, :], (LB2, 2 * OUT))
        o_ref[r] = mixed * wv2 + rowv


def kernel(series, peaks, variance, conv_kernel, conv_bias):
    B, L, C = series.shape
    P = peaks.shape[1]
    OUT = 2 * EMBED_DIMS
    feat = OUT - EMBED_DIMS - 1

    sc_scatter, rows_w = _make_sc_scatter(B, L, P)
    peaks_flat = peaks.reshape(B * P)
    rowoff = jnp.asarray(
        ((np.arange(B * P, dtype=np.int64) // P) % rows_w * L).astype(np.int32))
    delta = sc_scatter(peaks_flat, rowoff).reshape(B, L)

    RB = 64
    LB = 256
    NG = B // RB
    s3 = series.reshape(B, L // 2, 2)
    se, so = s3[:, :, 0], s3[:, :, 1]
    d3 = delta.reshape(B, L // 2, 2)
    de, do = d3[:, :, 0], d3[:, :, 1]
    v16 = jnp.broadcast_to(variance.reshape(B, 1), (B, _HALF))
    w64 = jnp.broadcast_to(
        jnp.concatenate([conv_kernel.reshape(feat),
                         jnp.zeros((OUT - feat,), jnp.float32)]), (8, OUT))
    b64 = jnp.broadcast_to(
        jnp.concatenate([conv_bias.reshape(feat),
                         jnp.zeros((OUT - feat,), jnp.float32)]), (8, OUT))

    out = pl.pallas_call(
        _tc_body,
        out_shape=jax.ShapeDtypeStruct((B, L // 2, 2 * OUT), jnp.float32),
        grid=(NG, L // LB),
        in_specs=[
            pl.BlockSpec((RB, LB // 2), lambda i, j: (i, j)),
            pl.BlockSpec((RB, LB // 2), lambda i, j: (i, j)),
            pl.BlockSpec((RB, LB // 2), lambda i, j: (i, j)),
            pl.BlockSpec((RB, LB // 2), lambda i, j: (i, j)),
            pl.BlockSpec((RB, _HALF), lambda i, j: (i, 0)),
            pl.BlockSpec((8, OUT), lambda i, j: (0, 0)),
            pl.BlockSpec((8, OUT), lambda i, j: (0, 0)),
        ],
        out_specs=pl.BlockSpec((RB, LB // 2, 2 * OUT), lambda i, j: (i, j, 0)),
        compiler_params=pltpu.CompilerParams(
            dimension_semantics=("arbitrary", "arbitrary")),
    )(se, so, de, do, v16, w64, b64)
    return out.reshape(B, L, OUT)


# speedup vs baseline: 11.1553x; 11.1553x over previous
import functools
import math

import numpy as np
import jax
import jax.numpy as jnp
from jax import lax
from jax.experimental import pallas as pl
from jax.experimental.pallas import tpu as pltpu
from jax.experimental.pallas import tpu_sc as plsc

EMBED_DIMS = 32
_HALF = EMBED_DIMS // 2


def _make_sc_scatter(B: int, L: int, P: int):
    info = plsc.get_sparse_core_info()
    NW = info.num_cores * info.num_subcores
    assert B % NW == 0
    rows_w = B // NW
    blk_len = rows_w * L
    pk_w = rows_w * P
    assert pk_w % 16 == 0 and pk_w % 8 == 0 and blk_len % 8 == 0
    n_vec = pk_w // 16

    mesh = plsc.VectorSubcoreMesh(core_axis_name="c", subcore_axis_name="s")

    @functools.partial(
        pl.kernel,
        out_type=jax.ShapeDtypeStruct((B * L,), jnp.float32),
        mesh=mesh,
        compiler_params=pltpu.CompilerParams(needs_layout_passes=False),
        scratch_types=[
            pltpu.VMEM((blk_len,), jnp.float32),
            pltpu.VMEM((pk_w,), jnp.int32),
            pltpu.VMEM((pk_w,), jnp.int32),
        ],
    )
    def sc_scatter(peaks_hbm, rowoff_hbm, out_hbm, blk, pk, ro):
        wid = lax.axis_index("s") * info.num_cores + lax.axis_index("c")

        pltpu.sync_copy(peaks_hbm.at[pl.ds(wid * pk_w, pk_w)], pk)
        pltpu.sync_copy(rowoff_hbm.at[pl.ds(wid * pk_w, pk_w)], ro)

        z16 = jnp.zeros((16,), jnp.float32)

        def zero_body(i, carry):
            base = i * 128
            for j in range(8):
                blk[pl.ds(base + j * 16, 16)] = z16
            return carry

        lax.fori_loop(0, blk_len // 128, zero_body, 0)

        ones16 = jnp.ones((16,), jnp.float32)

        def scat_body(j, carry):
            col = pk[pl.ds(j * 16, 16)]
            off = ro[pl.ds(j * 16, 16)]
            plsc.store_scatter(blk, [off + col], ones16, mask=col != 0)
            return carry

        lax.fori_loop(0, n_vec, scat_body, 0)

        pltpu.sync_copy(blk, out_hbm.at[pl.ds(wid * blk_len, blk_len)])

    return sc_scatter, rows_w


def _tc_body(s_ref, d_ref, v_ref, w_ref, b_ref, o_ref):
    RB, LBL = s_ref.shape
    OUT = o_ref.shape[1]
    feat = OUT - EMBED_DIMS - 1
    k16 = lax.broadcasted_iota(jnp.int32, (1, _HALF), 1).astype(jnp.float32)
    log_step = math.log(1000.0) / (_HALF - 1)
    ang_c = (2.0 * math.pi) * jnp.exp(k16 * log_step)
    av = v_ref[...] * ang_c
    row_all = b_ref[...][0:1, :] + jnp.concatenate(
        [jnp.zeros((RB, EMBED_DIMS), jnp.float32), jnp.sin(av), jnp.cos(av)],
        axis=1)
    wcol = lax.transpose(w_ref[...], (1, 0))[:, 0:1]
    rcolT = lax.transpose(row_all, (1, 0))
    wbc = jnp.broadcast_to(wcol, (OUT, LBL))
    frow = lax.broadcasted_iota(jnp.int32, (OUT, LBL), 0)
    is31 = frow == feat
    s_all = s_ref[...]
    d_all = d_ref[...]
    for r in range(RB):
        sbc = jnp.broadcast_to(s_all[r:r + 1, :], (OUT, LBL))
        dbc = jnp.broadcast_to(d_all[r:r + 1, :], (OUT, LBL))
        rbc = jnp.broadcast_to(rcolT[:, r:r + 1], (OUT, LBL))
        o_ref[r] = wbc * sbc + jnp.where(is31, dbc, rbc)


def kernel(series, peaks, variance, conv_kernel, conv_bias):
    B, L, C = series.shape
    P = peaks.shape[1]
    OUT = 2 * EMBED_DIMS
    feat = OUT - EMBED_DIMS - 1

    sc_scatter, rows_w = _make_sc_scatter(B, L, P)
    peaks_flat = peaks.reshape(B * P)
    rowoff = jnp.asarray(
        ((np.arange(B * P, dtype=np.int64) // P) % rows_w * L).astype(np.int32))
    delta = sc_scatter(peaks_flat, rowoff).reshape(B, L)

    RB = 64
    LB = 256
    NG = B // RB
    s2 = series.reshape(B, L)
    v16 = jnp.broadcast_to(variance.reshape(B, 1), (B, _HALF))
    w64 = jnp.broadcast_to(
        jnp.concatenate([conv_kernel.reshape(feat),
                         jnp.zeros((OUT - feat,), jnp.float32)]), (8, OUT))
    b64 = jnp.broadcast_to(
        jnp.concatenate([conv_bias.reshape(feat),
                         jnp.zeros((OUT - feat,), jnp.float32)]), (8, OUT))

    out_p = pl.pallas_call(
        _tc_body,
        out_shape=jax.ShapeDtypeStruct((B, OUT, L), jnp.float32),
        grid=(NG, L // LB),
        in_specs=[
            pl.BlockSpec((RB, LB), lambda i, j: (i, j)),
            pl.BlockSpec((RB, LB), lambda i, j: (i, j)),
            pl.BlockSpec((RB, _HALF), lambda i, j: (i, 0)),
            pl.BlockSpec((8, OUT), lambda i, j: (0, 0)),
            pl.BlockSpec((8, OUT), lambda i, j: (0, 0)),
        ],
        out_specs=pl.BlockSpec((RB, OUT, LB), lambda i, j: (i, 0, j)),
        compiler_params=pltpu.CompilerParams(
            dimension_semantics=("arbitrary", "arbitrary")),
    )(s2, delta, v16, w64, b64)
    return jnp.transpose(out_p, (0, 2, 1))
